# baseline (speedup 1.0000x reference)
import functools

import jax
import jax.numpy as jnp
from jax import lax
from jax.experimental import pallas as pl
from jax.experimental.pallas import tpu as pltpu
from jax.experimental.pallas import tpu_sc as plsc

VOCAB = 1000000
EMB = 64
NEGS = 15
BATCH = 16384

NC, NS, L = 2, 16, 16
NW = NC * NS
BPW = BATCH // NW
NG = BPW // 128
ROWS = NEGS + 1
NJ = EMB // L

_mesh = plsc.VectorSubcoreMesh(core_axis_name="c", subcore_axis_name="s")


@functools.partial(
    pl.kernel,
    out_type=jax.ShapeDtypeStruct((NW, ROWS, BPW), jnp.float32),
    mesh=_mesh,
    scratch_types=[
        pltpu.VMEM((NG, 128), jnp.int32),
        pltpu.VMEM((NG, 128), jnp.int32),
        pltpu.VMEM((NEGS * NG, 128), jnp.int32),
        pltpu.VMEM((BPW, EMB), jnp.float32),
        pltpu.VMEM((BPW, EMB), jnp.float32),
        pltpu.VMEM((BPW, EMB), jnp.float32),
        pltpu.VMEM((BPW * L,), jnp.float32),
        pltpu.VMEM((ROWS, BPW), jnp.float32),
        pltpu.SemaphoreType.DMA,
        pltpu.SemaphoreType.DMA,
        pltpu.SemaphoreType.DMA,
    ],
    compiler_params=pltpu.CompilerParams(needs_layout_passes=False,
                                         use_tc_tiling_on_sc=False),
)
def _sc_dots(emb_t, emb_c, t2, c2, n3, m2, out,
             tidx, cidx, nidx, ev, mv, nv, pv, sv, sem_e, sem_a, sem_b):
    wid = lax.axis_index("s") * NC + lax.axis_index("c")

    pltpu.sync_copy(t2.at[pl.ds(wid * NG, NG)], tidx)
    pltpu.sync_copy(c2.at[pl.ds(wid * NG, NG)], cidx)
    pltpu.sync_copy(n3.at[wid], nidx)

    eh = [pltpu.async_copy(emb_t.at[tidx.at[j]], ev.at[pl.ds(j * 128, 128)],
                           sem_e) for j in range(NG)]
    mh = pltpu.async_copy(m2.at[pl.ds(wid * BPW, BPW)], mv, sem_e)
    for h in eh:
        h.wait()
    mh.wait()

    ch = [pltpu.async_copy(emb_c.at[cidx.at[j]], nv.at[pl.ds(j * 128, 128)],
                           sem_a) for j in range(NG)]

    def _scale(b, carry):
        for j in range(NJ):
            sl = pl.ds(j * L, L)
            ev[b, sl] = ev[b, sl] * mv[b, sl]
        return carry

    lax.fori_loop(0, BPW, _scale, 0)

    lane16 = lax.iota(jnp.int32, L) * L

    def _dot_pass(nb, r):
        def _body(b, carry):
            p = ev[b, pl.ds(0, L)] * nb[b, pl.ds(0, L)]
            for j in range(1, NJ):
                sl = pl.ds(j * L, L)
                p = p + ev[b, sl] * nb[b, sl]
            pv[pl.ds(b * L, L)] = p
            return carry

        lax.fori_loop(0, BPW, _body, 0)

        def _red(g, carry):
            base = g * (L * L)
            acc = plsc.load_gather(pv, [lane16 + base])
            for d in range(1, L):
                acc = acc + plsc.load_gather(pv, [lane16 + (base + d)])
            sv[r, pl.ds(g * L, L)] = acc
            return carry

        lax.fori_loop(0, BPW // L, _red, 0)

    for h in ch:
        h.wait()

    bufs = [nv, mv]
    sems = [sem_a, sem_b]
    pend = [pltpu.async_copy(emb_c.at[nidx.at[j]],
                             mv.at[pl.ds(j * 128, 128)], sem_b)
            for j in range(NG)]

    _dot_pass(nv, 0)

    for k in range(NEGS):
        for h in pend:
            h.wait()
        cur = bufs[(k + 1) % 2]
        if k + 1 < NEGS:
            nxt = bufs[k % 2]
            pend = [pltpu.async_copy(emb_c.at[nidx.at[(k + 1) * NG + j]],
                                     nxt.at[pl.ds(j * 128, 128)],
                                     sems[k % 2])
                    for j in range(NG)]
        else:
            pend = []
        _dot_pass(cur, k + 1)

    pltpu.sync_copy(sv, out.at[wid])


def _tc_body(x_ref, o_ref):
    x = x_ref[...]
    sp = jnp.maximum(x, 0.0) + jnp.log1p(jnp.exp(-jnp.abs(x)))
    o_ref[...] = jnp.full((1, 1), jnp.sum(sp) * jnp.float32(1.0 / BATCH),
                          jnp.float32)


_tc_reduce = pl.pallas_call(
    _tc_body,
    out_shape=jax.ShapeDtypeStruct((1, 1), jnp.float32),
)


def kernel(emb_target, emb_context, target, context):
    key = jax.random.key(42)
    k_drop, k_neg = jax.random.split(key)
    keep = jax.random.bernoulli(k_drop, 0.9, (BATCH, EMB))
    mask = jnp.where(keep, jnp.float32(1.0 / 0.9), jnp.float32(0.0))
    neg = jax.random.randint(k_neg, (BATCH, NEGS), 0, VOCAB)

    t2 = target.astype(jnp.int32).reshape(BATCH // 128, 128)
    c2 = context.astype(jnp.int32).reshape(BATCH // 128, 128)
    negA = (neg.astype(jnp.int32).T
            .reshape(NEGS, NW, BPW).swapaxes(0, 1)
            .reshape(NW, NEGS * NG, 128))

    dots = _sc_dots(emb_target, emb_context, t2, c2, negA, mask)
    total = _tc_reduce(dots.reshape(BATCH * ROWS // 128, 128))
    return total[0, 0]


# speedup vs baseline: 1.1185x; 1.1185x over previous
import functools

import jax
import jax.numpy as jnp
from jax import lax
from jax.experimental import pallas as pl
from jax.experimental.pallas import tpu as pltpu
from jax.experimental.pallas import tpu_sc as plsc

VOCAB = 1000000
EMB = 64
NEGS = 15
BATCH = 16384

NC, NS, L = 2, 16, 16
NW = NC * NS
BPW = BATCH // NW
NG = BPW // 128
ROWS = NEGS + 1
NJ = EMB // L

_mesh = plsc.VectorSubcoreMesh(core_axis_name="c", subcore_axis_name="s")


@functools.partial(
    pl.kernel,
    out_type=jax.ShapeDtypeStruct((NW, ROWS, BPW), jnp.float32),
    mesh=_mesh,
    scratch_types=[
        pltpu.VMEM((NG, 128), jnp.int32),
        pltpu.VMEM((NG, 128), jnp.int32),
        pltpu.VMEM((NEGS * NG, 128), jnp.int32),
        pltpu.VMEM((BPW, EMB), jnp.float32),
        pltpu.VMEM((BPW, EMB), jnp.float32),
        pltpu.VMEM((BPW, EMB), jnp.float32),
        pltpu.VMEM((BPW * L,), jnp.float32),
        pltpu.VMEM((ROWS, BPW), jnp.float32),
        pltpu.SemaphoreType.DMA,
        pltpu.SemaphoreType.DMA,
        pltpu.SemaphoreType.DMA,
    ],
    compiler_params=pltpu.CompilerParams(needs_layout_passes=False,
                                         use_tc_tiling_on_sc=False),
)
def _sc_dots(emb_t, emb_c, t2, c2, n3, m2, out,
             tidx, cidx, nidx, ev, mv, nv, pv, sv, sem_e, sem_a, sem_b):
    wid = lax.axis_index("s") * NC + lax.axis_index("c")

    pltpu.sync_copy(t2.at[pl.ds(wid * NG, NG)], tidx)
    pltpu.sync_copy(c2.at[pl.ds(wid * NG, NG)], cidx)
    pltpu.sync_copy(n3.at[wid], nidx)

    eh = [pltpu.async_copy(emb_t.at[tidx.at[j]], ev.at[pl.ds(j * 128, 128)],
                           sem_e) for j in range(NG)]
    mh = pltpu.async_copy(m2.at[pl.ds(wid * BPW, BPW)], mv, sem_e)
    for h in eh:
        h.wait()
    mh.wait()

    ch = [pltpu.async_copy(emb_c.at[cidx.at[j]], nv.at[pl.ds(j * 128, 128)],
                           sem_a) for j in range(NG)]

    def _scale(b, carry):
        for j in range(NJ):
            sl = pl.ds(j * L, L)
            ev[b, sl] = ev[b, sl] * mv[b, sl]
        return carry

    lax.fori_loop(0, BPW, _scale, 0)

    lane16 = lax.iota(jnp.int32, L) * L

    def _dot_pass(nb, r):
        def _body(b, carry):
            p = ev[b, pl.ds(0, L)] * nb[b, pl.ds(0, L)]
            for j in range(1, NJ):
                sl = pl.ds(j * L, L)
                p = p + ev[b, sl] * nb[b, sl]
            pv[pl.ds(b * L, L)] = p
            return carry

        lax.fori_loop(0, BPW, _body, 0)

        def _red(g, carry):
            base = g * (L * L)
            acc = plsc.load_gather(pv, [lane16 + base])
            for d in range(1, L):
                acc = acc + plsc.load_gather(pv, [lane16 + (base + d)])
            sv[r, pl.ds(g * L, L)] = acc
            return carry

        lax.fori_loop(0, BPW // L, _red, 0)

    for h in ch:
        h.wait()

    bufs = [nv, mv]
    sems = [sem_a, sem_b]
    pend = [pltpu.async_copy(emb_c.at[nidx.at[j]],
                             mv.at[pl.ds(j * 128, 128)], sem_b)
            for j in range(NG)]

    _dot_pass(nv, 0)

    for k in range(NEGS):
        for h in pend:
            h.wait()
        cur = bufs[(k + 1) % 2]
        if k + 1 < NEGS:
            nxt = bufs[k % 2]
            pend = [pltpu.async_copy(emb_c.at[nidx.at[(k + 1) * NG + j]],
                                     nxt.at[pl.ds(j * 128, 128)],
                                     sems[k % 2])
                    for j in range(NG)]
        else:
            pend = []
        _dot_pass(cur, k + 1)

    pltpu.sync_copy(sv, out.at[wid])


_TRB = 2048


_TRG = (VOCAB + _TRB - 1) // _TRB
_TROWS = _TRG * _TRB


def _tr_body(x_ref, o_ref):
    y = jnp.transpose(x_ref[...])
    o_ref[...] = jnp.concatenate([y[: _TRB // 2], y[_TRB // 2:]], axis=1)


_tc_transpose = pl.pallas_call(
    _tr_body,
    grid=(_TRG,),
    in_specs=[pl.BlockSpec((EMB, _TRB), lambda i: (0, i))],
    out_specs=pl.BlockSpec((_TRB // 2, 2 * EMB), lambda i: (i, 0)),
    out_shape=jax.ShapeDtypeStruct((_TROWS // 2, 2 * EMB), jnp.float32),
)


def _flat_row(v):
    return (v & ~(_TRB - 1)) | ((v & (_TRB // 2 - 1)) << 1) | ((v >> 10) & 1)


def _tc_body(x_ref, o_ref):
    x = x_ref[...]
    sp = jnp.maximum(x, 0.0) + jnp.log1p(jnp.exp(-jnp.abs(x)))
    o_ref[...] = jnp.full((1, 1), jnp.sum(sp) * jnp.float32(1.0 / BATCH),
                          jnp.float32)


_tc_reduce = pl.pallas_call(
    _tc_body,
    out_shape=jax.ShapeDtypeStruct((1, 1), jnp.float32),
)


def kernel(emb_target, emb_context, target, context):
    key = jax.random.key(42)
    k_drop, k_neg = jax.random.split(key)
    keep = jax.random.bernoulli(k_drop, 0.9, (BATCH, EMB))
    mask = jnp.where(keep, jnp.float32(1.0 / 0.9), jnp.float32(0.0))
    neg = jax.random.randint(k_neg, (BATCH, NEGS), 0, VOCAB)

    t2 = _flat_row(target.astype(jnp.int32)).reshape(BATCH // 128, 128)
    c2 = _flat_row(context.astype(jnp.int32)).reshape(BATCH // 128, 128)
    negA = (_flat_row(neg.astype(jnp.int32)).T
            .reshape(NEGS, NW, BPW).swapaxes(0, 1)
            .reshape(NW, NEGS * NG, 128))

    tgt_rm = _tc_transpose(emb_target.T).reshape(_TROWS, EMB)
    ctx_rm = _tc_transpose(emb_context.T).reshape(_TROWS, EMB)
    dots = _sc_dots(tgt_rm, ctx_rm, t2, c2, negA, mask)
    total = _tc_reduce(dots.reshape(BATCH * ROWS // 128, 128))
    return total[0, 0]
